# Initial kernel scaffold; baseline (speedup 1.0000x reference)
import jax
import jax.numpy as jnp
from jax.experimental import pallas as pl


def kernel(x, edge_index, W_l, b_l, W_r):
    raise NotImplementedError("write your pallas kernel here")


# speedup vs baseline: 4.1527x; 4.1527x over previous
import functools

import jax
import jax.numpy as jnp
from jax import lax
from jax.experimental import pallas as pl
from jax.experimental.pallas import tpu as pltpu
from jax.experimental.pallas import tpu_sc as plsc

N_NODES = 10000
N_EDGES = 320000
D = 128

NC = 2
NS = 16
NW = NC * NS
CHUNK = 128
EPT = 10240
NCHUNK = EPT // CHUNK
E_PAD = NW * EPT
R_ACC = 10240
RPT = R_ACC // NS


def _sc_segment_sum(x, src_r, dst_r, z2d, z1d, ones_h):
    mesh = plsc.VectorSubcoreMesh(core_axis_name="c", subcore_axis_name="s")

    @functools.partial(
        pl.kernel,
        out_type=[
            jax.ShapeDtypeStruct((NC, R_ACC, D), jnp.float32),
            jax.ShapeDtypeStruct((NC, R_ACC), jnp.float32),
        ],
        mesh=mesh,
        scratch_types=[
            pltpu.VMEM((NCHUNK, CHUNK), jnp.int32),
            pltpu.VMEM((NCHUNK, CHUNK), jnp.int32),
            pltpu.VMEM((CHUNK, D), jnp.float32),
            pltpu.VMEM((CHUNK,), jnp.float32),
            pltpu.VMEM_SHARED((R_ACC, D), jnp.float32),
            pltpu.VMEM_SHARED((R_ACC,), jnp.float32),
            pltpu.SemaphoreType.DMA,
        ],
    )
    def k(x_hbm, src_hbm, dst_hbm, z2_hbm, z1_hbm, ones_hbm,
          sum_out, cnt_out, src_v, dst_v, rows_v, ones_v, acc_sh, cnt_sh,
          sem):
        c = lax.axis_index("c")
        s = lax.axis_index("s")
        wid = s * NC + c

        pltpu.sync_copy(z2_hbm, acc_sh.at[pl.ds(s * RPT, RPT)])
        pltpu.sync_copy(z1_hbm.at[pl.ds(s * RPT, RPT)],
                        cnt_sh.at[pl.ds(s * RPT, RPT)])
        pltpu.sync_copy(ones_hbm, ones_v)
        pltpu.sync_copy(src_hbm.at[pl.ds(wid * NCHUNK, NCHUNK)], src_v)
        pltpu.sync_copy(dst_hbm.at[pl.ds(wid * NCHUNK, NCHUNK)], dst_v)
        plsc.subcore_barrier()

        def chunk(j, carry):
            pltpu.async_copy(x_hbm.at[src_v.at[j]], rows_v, sem).wait()
            pltpu.sync_copy(rows_v, acc_sh.at[dst_v.at[j]], add=True)
            pltpu.sync_copy(ones_v, cnt_sh.at[dst_v.at[j]], add=True)
            return carry

        lax.fori_loop(0, NCHUNK, chunk, 0)
        plsc.subcore_barrier()

        pltpu.sync_copy(acc_sh.at[pl.ds(s * RPT, RPT)],
                        sum_out.at[c, pl.ds(s * RPT, RPT)])
        pltpu.sync_copy(cnt_sh.at[pl.ds(s * RPT, RPT)],
                        cnt_out.at[c, pl.ds(s * RPT, RPT)])

    return k(x, src_r, dst_r, z2d, z1d, ones_h)


def _tc_finish_body(p_ref, c_ref, x_ref, wl_ref, bl_ref, wr_ref, o_ref):
    summed = p_ref[0] + p_ref[1]
    cnt = c_ref[0] + c_ref[1]
    mean = summed / jnp.maximum(cnt, 1.0)[:, None]
    out = lax.dot_general(mean, wl_ref[...], (((1,), (1,)), ((), ())),
                          preferred_element_type=jnp.float32)
    out = out + lax.dot_general(x_ref[...], wr_ref[...],
                                (((1,), (1,)), ((), ())),
                                preferred_element_type=jnp.float32)
    out = out + bl_ref[...]
    nrm = jnp.sqrt(jnp.sum(out * out, axis=-1, keepdims=True))
    o_ref[...] = out / jnp.maximum(nrm, 1e-12)


def _tc_finish(partial_sum, partial_cnt, xp, W_l, b_l, W_r):
    RB = 128
    grid = (R_ACC // RB,)
    return pl.pallas_call(
        _tc_finish_body,
        grid=grid,
        in_specs=[
            pl.BlockSpec((NC, RB, D), lambda i: (0, i, 0)),
            pl.BlockSpec((NC, RB), lambda i: (0, i)),
            pl.BlockSpec((RB, D), lambda i: (i, 0)),
            pl.BlockSpec((D, D), lambda i: (0, 0)),
            pl.BlockSpec((1, D), lambda i: (0, 0)),
            pl.BlockSpec((D, D), lambda i: (0, 0)),
        ],
        out_specs=pl.BlockSpec((RB, D), lambda i: (i, 0)),
        out_shape=jax.ShapeDtypeStruct((R_ACC, D), jnp.float32),
    )(partial_sum, partial_cnt, xp, W_l, b_l, W_r)


def kernel(x, edge_index, W_l, b_l, W_r):
    src = edge_index[0].astype(jnp.int32)
    dst = edge_index[1].astype(jnp.int32)
    pad = E_PAD - N_EDGES
    src_r = jnp.concatenate(
        [src, jnp.zeros((pad,), jnp.int32)]).reshape(E_PAD // CHUNK, CHUNK)
    dst_r = jnp.concatenate(
        [dst, jnp.full((pad,), N_NODES, jnp.int32)]).reshape(
            E_PAD // CHUNK, CHUNK)
    z2d = jnp.zeros((RPT, D), jnp.float32)
    z1d = jnp.zeros((R_ACC,), jnp.float32)
    ones_h = jnp.ones((CHUNK,), jnp.float32)

    partial_sum, partial_cnt = _sc_segment_sum(
        x, src_r, dst_r, z2d, z1d, ones_h)

    xp = jnp.pad(x, ((0, R_ACC - N_NODES), (0, 0)))
    out = _tc_finish(partial_sum, partial_cnt, xp, W_l,
                     b_l.reshape(1, D), W_r)
    return out[:N_NODES]


# speedup vs baseline: 4.7836x; 1.1519x over previous
import functools

import jax
import jax.numpy as jnp
from jax import lax
from jax.experimental import pallas as pl
from jax.experimental.pallas import tpu as pltpu
from jax.experimental.pallas import tpu_sc as plsc

N_NODES = 10000
N_EDGES = 320000
D = 128

NC = 2
NS = 16
NW = NC * NS
CHUNK = 128
EPT = 10240
NCHUNK = EPT // CHUNK
NBUF = 2
NHALF = NCHUNK // 2
NSTEP = NHALF // NBUF
E_PAD = NW * EPT
R_ACC = 10240
RPT = R_ACC // NS


def _sc_segment_sum(x, src_r, dst_r, z2d, z1d, ones_h):
    mesh = plsc.VectorSubcoreMesh(core_axis_name="c", subcore_axis_name="s")

    @functools.partial(
        pl.kernel,
        out_type=[
            jax.ShapeDtypeStruct((NC, R_ACC, D), jnp.float32),
            jax.ShapeDtypeStruct((NC, R_ACC), jnp.float32),
        ],
        mesh=mesh,
        scratch_types=[
            pltpu.VMEM((NHALF, CHUNK), jnp.int32),
            pltpu.VMEM((NHALF, CHUNK), jnp.int32),
            pltpu.VMEM((NBUF, CHUNK, D), jnp.float32),
            pltpu.VMEM((CHUNK,), jnp.float32),
            pltpu.VMEM_SHARED((R_ACC, D), jnp.float32),
            pltpu.VMEM_SHARED((R_ACC,), jnp.float32),
            pltpu.SemaphoreType.DMA((NBUF,)),
            pltpu.SemaphoreType.DMA((NBUF,)),
            pltpu.SemaphoreType.DMA((NBUF,)),
        ],
    )
    def k(x_hbm, src_hbm, dst_hbm, z2_hbm, z1_hbm, ones_hbm,
          sum_out, cnt_out, src_v, dst_v, rows_v, ones_v, acc_sh, cnt_sh,
          gsem, ssem, csem):
        c = lax.axis_index("c")
        s = lax.axis_index("s")
        wid = s * NC + c

        pltpu.sync_copy(z2_hbm, acc_sh.at[pl.ds(s * RPT, RPT)])
        pltpu.sync_copy(z1_hbm.at[pl.ds(s * RPT, RPT)],
                        cnt_sh.at[pl.ds(s * RPT, RPT)])
        pltpu.sync_copy(ones_hbm, ones_v)
        plsc.subcore_barrier()

        def fire_gather(i, b):
            pltpu.async_copy(x_hbm.at[src_v.at[i]], rows_v.at[b],
                             gsem.at[b])

        def wait_gather(i, b):
            pltpu.make_async_copy(x_hbm.at[src_v.at[i]], rows_v.at[b],
                                  gsem.at[b]).wait()

        def fire_scatter(i, b):
            pltpu.async_copy(rows_v.at[b], acc_sh.at[dst_v.at[i]],
                             ssem.at[b], add=True)
            pltpu.async_copy(ones_v, cnt_sh.at[dst_v.at[i]],
                             csem.at[b], add=True)

        def wait_scatter(i, b):
            pltpu.make_async_copy(rows_v.at[b], acc_sh.at[dst_v.at[i]],
                                  ssem.at[b]).wait()
            pltpu.make_async_copy(ones_v, cnt_sh.at[dst_v.at[i]],
                                  csem.at[b]).wait()

        for p in range(2):
            base = wid * NCHUNK + p * NHALF
            pltpu.sync_copy(src_hbm.at[pl.ds(base, NHALF)], src_v)
            pltpu.sync_copy(dst_hbm.at[pl.ds(base, NHALF)], dst_v)

            for b in range(NBUF):
                fire_gather(b, b)

            def step(kk, carry):
                for b in range(NBUF):
                    i = kk * NBUF + b
                    wait_gather(i, b)
                    fire_scatter(i, b)
                    wait_scatter(i, b)
                    fire_gather(i + NBUF, b)
                return carry

            lax.fori_loop(0, NSTEP - 1, step, 0)
            for b in range(NBUF):
                i = (NSTEP - 1) * NBUF + b
                wait_gather(i, b)
                fire_scatter(i, b)
                wait_scatter(i, b)
        plsc.subcore_barrier()

        pltpu.sync_copy(acc_sh.at[pl.ds(s * RPT, RPT)],
                        sum_out.at[c, pl.ds(s * RPT, RPT)])
        pltpu.sync_copy(cnt_sh.at[pl.ds(s * RPT, RPT)],
                        cnt_out.at[c, pl.ds(s * RPT, RPT)])

    return k(x, src_r, dst_r, z2d, z1d, ones_h)


def _tc_finish_body(p_ref, c_ref, x_ref, wl_ref, bl_ref, wr_ref, o_ref):
    summed = p_ref[0] + p_ref[1]
    cnt = c_ref[0] + c_ref[1]
    mean = summed / jnp.maximum(cnt, 1.0)[:, None]
    out = lax.dot_general(mean, wl_ref[...], (((1,), (1,)), ((), ())),
                          preferred_element_type=jnp.float32)
    out = out + lax.dot_general(x_ref[...], wr_ref[...],
                                (((1,), (1,)), ((), ())),
                                preferred_element_type=jnp.float32)
    out = out + bl_ref[...]
    nrm = jnp.sqrt(jnp.sum(out * out, axis=-1, keepdims=True))
    o_ref[...] = out / jnp.maximum(nrm, 1e-12)


def _tc_finish(partial_sum, partial_cnt, xp, W_l, b_l, W_r):
    RB = 128
    grid = (R_ACC // RB,)
    return pl.pallas_call(
        _tc_finish_body,
        grid=grid,
        in_specs=[
            pl.BlockSpec((NC, RB, D), lambda i: (0, i, 0)),
            pl.BlockSpec((NC, RB), lambda i: (0, i)),
            pl.BlockSpec((RB, D), lambda i: (i, 0)),
            pl.BlockSpec((D, D), lambda i: (0, 0)),
            pl.BlockSpec((1, D), lambda i: (0, 0)),
            pl.BlockSpec((D, D), lambda i: (0, 0)),
        ],
        out_specs=pl.BlockSpec((RB, D), lambda i: (i, 0)),
        out_shape=jax.ShapeDtypeStruct((R_ACC, D), jnp.float32),
    )(partial_sum, partial_cnt, xp, W_l, b_l, W_r)


def kernel(x, edge_index, W_l, b_l, W_r):
    src = edge_index[0].astype(jnp.int32)
    dst = edge_index[1].astype(jnp.int32)
    pad = E_PAD - N_EDGES
    src_r = jnp.concatenate(
        [src, jnp.zeros((pad,), jnp.int32)]).reshape(E_PAD // CHUNK, CHUNK)
    dst_pad = N_NODES + (jnp.arange(pad, dtype=jnp.int32) % (R_ACC - N_NODES))
    dst_r = jnp.concatenate([dst, dst_pad]).reshape(E_PAD // CHUNK, CHUNK)
    z2d = jnp.zeros((RPT, D), jnp.float32)
    z1d = jnp.zeros((R_ACC,), jnp.float32)
    ones_h = jnp.ones((CHUNK,), jnp.float32)

    partial_sum, partial_cnt = _sc_segment_sum(
        x, src_r, dst_r, z2d, z1d, ones_h)

    xp = jnp.pad(x, ((0, R_ACC - N_NODES), (0, 0)))
    out = _tc_finish(partial_sum, partial_cnt, xp, W_l,
                     b_l.reshape(1, D), W_r)
    return out[:N_NODES]
